# baseline (speedup 1.0000x reference)
import functools

import jax
import jax.numpy as jnp
from jax import lax
from jax.experimental import pallas as pl
from jax.experimental.pallas import tpu as pltpu
from jax.experimental.pallas import tpu_sc as plsc

VOCAB = 100000
EMBED = 64
CTX = 50
HID = 128

VBLK = 2000
NBLK = VOCAB // VBLK

_N_WORKERS = 8
_ROWS_PER_W = 8
_PAD_B = _N_WORKERS * _ROWS_PER_W


def _sc_gather_body(idx_hbm, table_hbm, out_hbm, idx_v, rows_v, sem):
    wid = lax.axis_index("s") * 2 + lax.axis_index("c")

    @pl.when(wid < _N_WORKERS)
    def _():
        base = wid * _ROWS_PER_W
        pltpu.sync_copy(idx_hbm.at[pl.ds(base, _ROWS_PER_W)], idx_v)
        pltpu.async_copy(table_hbm.at[idx_v], rows_v, sem).wait()
        pltpu.sync_copy(rows_v, out_hbm.at[pl.ds(base, _ROWS_PER_W)])


@functools.cache
def _sc_gather():
    return pl.kernel(
        _sc_gather_body,
        out_type=jax.ShapeDtypeStruct((_PAD_B, EMBED), jnp.float32),
        mesh=plsc.VectorSubcoreMesh(core_axis_name="c", subcore_axis_name="s"),
        scratch_types=[
            pltpu.VMEM((_ROWS_PER_W,), jnp.int32),
            pltpu.VMEM((_ROWS_PER_W, EMBED), jnp.float32),
            pltpu.SemaphoreType.DMA,
        ],
        compiler_params=pltpu.CompilerParams(use_tc_tiling_on_sc=False),
    )


def _mlp_body(emb_ref, w1_ref, b1_ref, w2_ref, b2_ref, out_ref,
              m_ref, s_ref, h_ref):
    j = pl.program_id(0)

    @pl.when(j == 0)
    def _():
        h = lax.dot_general(
            emb_ref[...], w1_ref[...], (((1,), (1,)), ((), ())),
            preferred_element_type=jnp.float32)
        h_ref[...] = jnp.maximum(h + b1_ref[...], 0.0)
        m_ref[0] = -jnp.inf
        s_ref[0] = 0.0

    logits = lax.dot_general(
        h_ref[...], w2_ref[0], (((1,), (1,)), ((), ())),
        preferred_element_type=jnp.float32) + b2_ref[0]
    out_ref[pl.ds(j, 1)] = logits[None]

    bm = jnp.max(logits)
    m_old = m_ref[0]
    m_new = jnp.maximum(m_old, bm)
    s_ref[0] = s_ref[0] * jnp.exp(m_old - m_new) + jnp.sum(jnp.exp(logits - m_new))
    m_ref[0] = m_new

    @pl.when(j == NBLK - 1)
    def _():
        lse = m_ref[0] + jnp.log(s_ref[0])
        out_ref[...] = out_ref[...] - lse


_mlp_call = pl.pallas_call(
    _mlp_body,
    grid=(NBLK,),
    in_specs=[
        pl.BlockSpec((1, CTX * EMBED), lambda j: (0, 0)),
        pl.BlockSpec((HID, CTX * EMBED), lambda j: (0, 0)),
        pl.BlockSpec((1, HID), lambda j: (0, 0)),
        pl.BlockSpec((1, VBLK, HID), lambda j: (j, 0, 0)),
        pl.BlockSpec((1, 1, VBLK), lambda j: (j, 0, 0)),
    ],
    out_specs=pl.BlockSpec((NBLK, 1, VBLK), lambda j: (0, 0, 0)),
    out_shape=jax.ShapeDtypeStruct((NBLK, 1, VBLK), jnp.float32),
    scratch_shapes=[
        pltpu.SMEM((1,), jnp.float32),
        pltpu.SMEM((1,), jnp.float32),
        pltpu.VMEM((1, HID), jnp.float32),
    ],
    compiler_params=pltpu.CompilerParams(
        dimension_semantics=("arbitrary",)),
)


def kernel(inputs, emb_table, W1, b1, W2, b2):
    idx = jnp.zeros((_PAD_B,), jnp.int32).at[:CTX].set(inputs.astype(jnp.int32))
    rows = _sc_gather()(idx, emb_table)
    emb_flat = rows[:CTX].reshape(1, CTX * EMBED)
    out = _mlp_call(
        emb_flat,
        W1,
        b1.reshape(1, HID),
        W2.reshape(NBLK, VBLK, HID),
        b2.reshape(NBLK, 1, VBLK),
    )
    return out.reshape(1, VOCAB)


# speedup vs baseline: 1.4396x; 1.4396x over previous
import jax
import jax.numpy as jnp
from jax import lax
from jax.experimental import pallas as pl
from jax.experimental.pallas import tpu as pltpu

VOCAB = 100000
EMBED = 64
CTX = 50
HID = 128

VBLK = 2000
NBLK = VOCAB // VBLK


def _body(idx_ref, table_ref, w1_ref, b1_ref, w2_ref, b2_ref, out_ref,
          sem, emb_ref, h_ref, m_ref, s_ref):
    j = pl.program_id(0)

    @pl.when(j == 0)
    def _():
        copies = [
            pltpu.make_async_copy(
                table_ref.at[pl.ds(idx_ref[t], 1), :],
                emb_ref.at[pl.ds(t, 1), :],
                sem,
            )
            for t in range(CTX)
        ]
        for c in copies:
            c.start()
        for c in copies:
            c.wait()
        acc = jnp.zeros((1, HID), jnp.float32)
        for t in range(CTX):
            acc = acc + lax.dot_general(
                emb_ref[t:t + 1, :], w1_ref[:, t * EMBED:(t + 1) * EMBED],
                (((1,), (1,)), ((), ())), preferred_element_type=jnp.float32)
        h_ref[...] = jnp.maximum(acc + b1_ref[...], 0.0)
        m_ref[0] = -jnp.inf
        s_ref[0] = 0.0

    logits = lax.dot_general(
        h_ref[...], w2_ref[0], (((1,), (1,)), ((), ())),
        preferred_element_type=jnp.float32) + b2_ref[0]
    out_ref[pl.ds(j, 1)] = logits[None]

    bm = jnp.max(logits)
    m_old = m_ref[0]
    m_new = jnp.maximum(m_old, bm)
    s_ref[0] = s_ref[0] * jnp.exp(m_old - m_new) + jnp.sum(jnp.exp(logits - m_new))
    m_ref[0] = m_new

    @pl.when(j == NBLK - 1)
    def _():
        lse = m_ref[0] + jnp.log(s_ref[0])
        out_ref[...] = out_ref[...] - lse


_call = pl.pallas_call(
    _body,
    grid=(NBLK,),
    in_specs=[
        pl.BlockSpec(memory_space=pltpu.SMEM),
        pl.BlockSpec(memory_space=pl.ANY),
        pl.BlockSpec((HID, CTX * EMBED), lambda j: (0, 0)),
        pl.BlockSpec((1, HID), lambda j: (0, 0)),
        pl.BlockSpec((1, VBLK, HID), lambda j: (j, 0, 0)),
        pl.BlockSpec((1, 1, VBLK), lambda j: (j, 0, 0)),
    ],
    out_specs=pl.BlockSpec((NBLK, 1, VBLK), lambda j: (0, 0, 0)),
    out_shape=jax.ShapeDtypeStruct((NBLK, 1, VBLK), jnp.float32),
    scratch_shapes=[
        pltpu.SemaphoreType.DMA,
        pltpu.VMEM((CTX, EMBED), jnp.float32),
        pltpu.VMEM((1, HID), jnp.float32),
        pltpu.SMEM((1,), jnp.float32),
        pltpu.SMEM((1,), jnp.float32),
    ],
    compiler_params=pltpu.CompilerParams(
        dimension_semantics=("arbitrary",)),
)


def kernel(inputs, emb_table, W1, b1, W2, b2):
    out = _call(
        inputs.astype(jnp.int32),
        emb_table,
        W1,
        b1.reshape(1, HID),
        W2.reshape(NBLK, VBLK, HID),
        b2.reshape(NBLK, 1, VBLK),
    )
    return out.reshape(1, VOCAB)


# speedup vs baseline: 1.5244x; 1.0588x over previous
import jax
import jax.numpy as jnp
from jax import lax
from jax.experimental import pallas as pl
from jax.experimental.pallas import tpu as pltpu

VOCAB = 100000
EMBED = 64
CTX = 50
HID = 128

VBLK = 2000
NBLK = VOCAB // VBLK


def _body(idx_ref, table_ref, w1_ref, b1_ref, w2_ref, b2_ref, out_ref,
          sem, emb_ref, h_ref):
    j = pl.program_id(0)

    @pl.when(j == 0)
    def _():
        copies = [
            pltpu.make_async_copy(
                table_ref.at[pl.ds(idx_ref[t], 1), :],
                emb_ref.at[pl.ds(t, 1), :],
                sem,
            )
            for t in range(CTX)
        ]
        for c in copies:
            c.start()
        for c in copies:
            c.wait()
        acc = jnp.zeros((1, HID), jnp.float32)
        for t in range(CTX):
            acc = acc + lax.dot_general(
                emb_ref[t:t + 1, :].astype(jnp.bfloat16),
                w1_ref[:, t * EMBED:(t + 1) * EMBED].astype(jnp.bfloat16),
                (((1,), (1,)), ((), ())), preferred_element_type=jnp.float32)
        h_ref[...] = jnp.maximum(acc + b1_ref[...], 0.0).astype(jnp.bfloat16)

    logits = lax.dot_general(
        h_ref[...], w2_ref[0].astype(jnp.bfloat16),
        (((1,), (1,)), ((), ())),
        preferred_element_type=jnp.float32) + b2_ref[0]
    out_ref[pl.ds(j, 1)] = logits[None]

    @pl.when(j == NBLK - 1)
    def _():
        x = out_ref[...]
        m = jnp.max(x)
        lse = m + jnp.log(jnp.sum(jnp.exp(x - m)))
        out_ref[...] = x - lse


_call = pl.pallas_call(
    _body,
    grid=(NBLK,),
    in_specs=[
        pl.BlockSpec(memory_space=pltpu.SMEM),
        pl.BlockSpec(memory_space=pl.ANY),
        pl.BlockSpec((HID, CTX * EMBED), lambda j: (0, 0)),
        pl.BlockSpec((1, HID), lambda j: (0, 0)),
        pl.BlockSpec((1, VBLK, HID), lambda j: (j, 0, 0)),
        pl.BlockSpec((1, 1, VBLK), lambda j: (j, 0, 0)),
    ],
    out_specs=pl.BlockSpec((NBLK, 1, VBLK), lambda j: (0, 0, 0)),
    out_shape=jax.ShapeDtypeStruct((NBLK, 1, VBLK), jnp.float32),
    scratch_shapes=[
        pltpu.SemaphoreType.DMA,
        pltpu.VMEM((CTX, EMBED), jnp.float32),
        pltpu.VMEM((1, HID), jnp.bfloat16),
    ],
    compiler_params=pltpu.CompilerParams(
        dimension_semantics=("arbitrary",)),
)


def kernel(inputs, emb_table, W1, b1, W2, b2):
    out = _call(
        inputs.astype(jnp.int32),
        emb_table,
        W1,
        b1.reshape(1, HID),
        W2.reshape(NBLK, VBLK, HID),
        b2.reshape(NBLK, 1, VBLK),
    )
    return out.reshape(1, VOCAB)


# speedup vs baseline: 2.0185x; 1.3241x over previous
import jax
import jax.numpy as jnp
from jax import lax
from jax.experimental import pallas as pl
from jax.experimental.pallas import tpu as pltpu

VOCAB = 100000
EMBED = 64
CTX = 50
HID = 128

VBLK = 2000
NSTREAM = 5
NBLK = VOCAB // VBLK
NSTEP = NBLK // NSTREAM


def _body(idx_ref, table_ref, w1_ref, b1_ref, *rest):
    w2_refs = rest[:NSTREAM]
    b2_ref, out_ref, sem, emb_ref, h_ref = rest[NSTREAM:]
    j = pl.program_id(0)

    @pl.when(j == 0)
    def _():
        copies = [
            pltpu.make_async_copy(
                table_ref.at[pl.ds(idx_ref[t], 1), :],
                emb_ref.at[pl.ds(t, 1), :],
                sem,
            )
            for t in range(CTX)
        ]
        for c in copies:
            c.start()
        for c in copies:
            c.wait()
        acc = jnp.zeros((1, HID), jnp.float32)
        for t in range(CTX):
            acc = acc + lax.dot_general(
                emb_ref[t:t + 1, :].astype(jnp.bfloat16),
                w1_ref[:, t * EMBED:(t + 1) * EMBED].astype(jnp.bfloat16),
                (((1,), (1,)), ((), ())), preferred_element_type=jnp.float32)
        h_ref[...] = jnp.maximum(acc + b1_ref[...], 0.0).astype(jnp.bfloat16)

    for g in range(NSTREAM):
        logits = lax.dot_general(
            h_ref[...], w2_refs[g][0].astype(jnp.bfloat16),
            (((1,), (1,)), ((), ())),
            preferred_element_type=jnp.float32,
        ) + b2_ref[:, 0, g * VBLK:(g + 1) * VBLK]
        out_ref[pl.ds(j * NSTREAM + g, 1)] = logits[None]

    @pl.when(j == NSTEP - 1)
    def _():
        x = out_ref[...]
        m = jnp.max(x)
        lse = m + jnp.log(jnp.sum(jnp.exp(x - m)))
        out_ref[...] = x - lse


def _w2_spec(g):
    return pl.BlockSpec((1, VBLK, HID), lambda j, g=g: (j * NSTREAM + g, 0, 0))


_call = pl.pallas_call(
    _body,
    grid=(NSTEP,),
    in_specs=[
        pl.BlockSpec(memory_space=pltpu.SMEM),
        pl.BlockSpec(memory_space=pl.ANY),
        pl.BlockSpec((HID, CTX * EMBED), lambda j: (0, 0)),
        pl.BlockSpec((1, HID), lambda j: (0, 0)),
    ] + [_w2_spec(g) for g in range(NSTREAM)] + [
        pl.BlockSpec((1, 1, NSTREAM * VBLK), lambda j: (j, 0, 0)),
    ],
    out_specs=pl.BlockSpec((NBLK, 1, VBLK), lambda j: (0, 0, 0)),
    out_shape=jax.ShapeDtypeStruct((NBLK, 1, VBLK), jnp.float32),
    scratch_shapes=[
        pltpu.SemaphoreType.DMA,
        pltpu.VMEM((CTX, EMBED), jnp.float32),
        pltpu.VMEM((1, HID), jnp.bfloat16),
    ],
    compiler_params=pltpu.CompilerParams(
        dimension_semantics=("arbitrary",)),
)


def kernel(inputs, emb_table, W1, b1, W2, b2):
    w2r = W2.reshape(NBLK, VBLK, HID)
    out = _call(
        inputs.astype(jnp.int32),
        emb_table,
        W1,
        b1.reshape(1, HID),
        *([w2r] * NSTREAM),
        b2.reshape(NSTEP, 1, NSTREAM * VBLK),
    )
    return out.reshape(1, VOCAB)
